# baseline (speedup 1.0000x reference)
import jax
import jax.numpy as jnp
from jax.experimental import pallas as pl


def _final_add(a_ref, b_ref, o_ref):
    o_ref[...] = a_ref[...] + b_ref[...]


def _gcn(h, W, b, src, dst, ew):
    n = h.shape[0]
    loop = jnp.arange(n, dtype=src.dtype)
    s = jnp.concatenate([src, loop])
    d = jnp.concatenate([dst, loop])
    w = jnp.concatenate([ew, jnp.ones((n,), dtype=ew.dtype)])
    deg = jnp.zeros((n,), dtype=ew.dtype).at[d].add(w)
    dis = jnp.where(deg > 0, jax.lax.rsqrt(jnp.maximum(deg, 1e-12)), 0.0)
    norm = dis[s] * w * dis[d]
    m = h @ W
    msg = m[s] * norm[:, None]
    out = jnp.zeros((n, W.shape[1]), dtype=m.dtype).at[d].add(msg)
    return out, b


def kernel(x, edge_index, edge_attr, CE, SE, W1, b1, W2, b2):
    country_emb = CE[x[:, 0]]
    sector_emb = SE[x[:, 1]]
    h = jnp.concatenate([country_emb, sector_emb], axis=1)
    src = edge_index[0]
    dst = edge_index[1]
    h, bb = _gcn(h, W1, b1, src, dst, edge_attr)
    h = jax.nn.relu(h + bb)
    h, bb = _gcn(h, W2, b2, src, dst, edge_attr)
    out = pl.pallas_call(
        _final_add,
        grid=(50,),
        in_specs=[pl.BlockSpec((1000, 16), lambda i: (i, 0))] * 2,
        out_specs=pl.BlockSpec((1000, 16), lambda i: (i, 0)),
        out_shape=jax.ShapeDtypeStruct(h.shape, h.dtype),
    )(h, jnp.broadcast_to(bb, h.shape))
    return out


# speedup vs baseline: 20.1244x; 20.1244x over previous
import functools

import jax
import jax.numpy as jnp
from jax import lax
from jax.experimental import pallas as pl
from jax.experimental.pallas import tpu as pltpu, tpu_sc as plsc

N = 50000
E = 800000
EM = 128
ER = 6256
EP = ER * EM
CH = 8
NCHUNK = ER // CH
NPAD = 49 * 1024
NC, NS = 2, 16
NW = NC * NS
BLK = 1000
GRID = N // BLK

_mesh = functools.partial(
    plsc.VectorSubcoreMesh, core_axis_name="c", subcore_axis_name="s",
    num_cores=NC, num_subcores=NS)


def _zero16():
    return jnp.zeros((16,), jnp.float32)


def _deg_body(dst_hbm, ew_hbm, out_hbm, acc, zbuf, dbuf, wbuf, sem):
    core = lax.axis_index("c")
    sid = lax.axis_index("s")
    wid = sid * NC + core

    def zb(i, _):
        zbuf[pl.ds(i * 16, 16)] = _zero16()
        return 0
    lax.fori_loop(0, 64, zb, 0)
    for z in range(4):
        cid = sid + z * NS
        @pl.when(cid < NPAD // 1024)
        def _():
            pltpu.sync_copy(zbuf, acc.at[pl.ds(cid * 1024, 1024)])
    plsc.subcore_barrier()

    def chunk(c, _):
        cid = c * NW + wid
        @pl.when(cid < NCHUNK)
        def _():
            r0 = cid * CH
            pltpu.sync_copy(dst_hbm.at[pl.ds(r0, CH)], dbuf)
            pltpu.sync_copy(ew_hbm.at[pl.ds(r0, CH)], wbuf)
            for j in range(CH):
                pltpu.sync_copy(wbuf.at[j], acc.at[dbuf.at[j]], add=True)
        return 0
    lax.fori_loop(0, (NCHUNK + NW - 1) // NW, chunk, 0)
    plsc.subcore_barrier()

    for z in range(4):
        cid = sid + z * NS
        @pl.when(cid < 48)
        def _():
            pltpu.sync_copy(acc.at[pl.ds(cid * 1024, 1024)], zbuf)
            pltpu.sync_copy(zbuf, out_hbm.at[pl.ds(core * N + cid * 1024, 1024)])
        @pl.when(cid == 48)
        def _():
            pltpu.sync_copy(acc.at[pl.ds(48 * 1024, 848)], zbuf.at[pl.ds(0, 848)])
            pltpu.sync_copy(zbuf.at[pl.ds(0, 848)],
                            out_hbm.at[pl.ds(core * N + 48 * 1024, 848)])


def _deg_pass(dst2, ew2):
    return pl.kernel(
        _deg_body,
        out_type=jax.ShapeDtypeStruct((NC * N,), jnp.float32),
        mesh=_mesh(),
        compiler_params=pltpu.CompilerParams(needs_layout_passes=False, use_tc_tiling_on_sc=False),
        scratch_types=[
            pltpu.VMEM_SHARED((NPAD,), jnp.float32),
            pltpu.VMEM((1024,), jnp.float32),
            pltpu.VMEM((CH, EM), jnp.int32),
            pltpu.VMEM((CH, EM), jnp.float32),
            pltpu.SemaphoreType.DMA,
        ],
    )(dst2, ew2)


def _normw_body(dis_hbm, dst_hbm, ew_hbm, out_hbm, dis_v, dbuf, wbuf, obuf):
    core = lax.axis_index("c")
    sid = lax.axis_index("s")
    wid = sid * NC + core

    pltpu.sync_copy(dis_hbm, dis_v)

    def chunk(c, _):
        cid = c * NW + wid
        @pl.when(cid < NCHUNK)
        def _():
            r0 = cid * CH
            pltpu.sync_copy(dst_hbm.at[pl.ds(r0, CH)], dbuf)
            pltpu.sync_copy(ew_hbm.at[pl.ds(r0, CH)], wbuf)

            def scale16(q, _):
                j = q // (EM // 16)
                o = (q % (EM // 16)) * 16
                d16 = dbuf[j, pl.ds(o, 16)]
                dval = plsc.load_gather(dis_v, [d16])
                obuf[j, pl.ds(o, 16)] = wbuf[j, pl.ds(o, 16)] * dval
                return 0
            lax.fori_loop(0, CH * EM // 16, scale16, 0)
            pltpu.sync_copy(obuf, out_hbm.at[pl.ds(r0, CH)])
        return 0
    lax.fori_loop(0, (NCHUNK + NW - 1) // NW, chunk, 0)


def _normw_pass(dis, dst2, ew2):
    return pl.kernel(
        _normw_body,
        out_type=jax.ShapeDtypeStruct((ER, EM), jnp.float32),
        mesh=_mesh(),
        compiler_params=pltpu.CompilerParams(needs_layout_passes=False, use_tc_tiling_on_sc=False),
        scratch_types=[
            pltpu.VMEM((N,), jnp.float32),
            pltpu.VMEM((CH, EM), jnp.int32),
            pltpu.VMEM((CH, EM), jnp.float32),
            pltpu.VMEM((CH, EM), jnp.float32),
        ],
    )(dis, dst2, ew2)


def _agg_body(D, CHD, g_hbm, src_hbm, dst_hbm, nw_hbm, out_hbm,
              acc, rows, sbuf, dbuf, wbuf, sem):
    core = lax.axis_index("c")
    sid = lax.axis_index("s")
    wid = sid * NC + core
    nh = D // 16
    zrows = CHD * EM
    nz = NPAD // zrows
    nchunk = ER // CHD

    def zr(r, _):
        for h in range(nh):
            rows[r, pl.ds(h * 16, 16)] = _zero16()
        return 0
    lax.fori_loop(0, zrows, zr, 0)
    for z in range((nz + NS - 1) // NS):
        cid = sid + z * NS
        @pl.when(cid < nz)
        def _():
            pltpu.sync_copy(rows, acc.at[pl.ds(cid * zrows, zrows)])
    plsc.subcore_barrier()

    def chunk(c, _):
        cid = c * NW + wid
        @pl.when(cid < nchunk)
        def _():
            r0 = cid * CHD
            pltpu.sync_copy(src_hbm.at[pl.ds(r0, CHD)], sbuf)
            pltpu.sync_copy(dst_hbm.at[pl.ds(r0, CHD)], dbuf)
            pltpu.sync_copy(nw_hbm.at[pl.ds(r0, CHD)], wbuf)
            for j in range(CHD):
                pltpu.async_copy(g_hbm.at[sbuf.at[j]],
                                 rows.at[pl.ds(j * EM, EM)], sem).wait()

            def edge(e, _):
                i0 = jnp.broadcast_to(e // EM, (16,))
                i1 = jnp.broadcast_to(e % EM, (16,))
                s16 = plsc.load_gather(wbuf, [i0, i1])
                for h in range(nh):
                    rows[e, pl.ds(h * 16, 16)] = rows[e, pl.ds(h * 16, 16)] * s16
                return 0
            lax.fori_loop(0, CHD * EM, edge, 0)

            for j in range(CHD):
                pltpu.sync_copy(rows.at[pl.ds(j * EM, EM)],
                                acc.at[dbuf.at[j]], add=True)
        return 0
    lax.fori_loop(0, (nchunk + NW - 1) // NW, chunk, 0)
    plsc.subcore_barrier()

    for z in range(7):
        cid = sid + z * NS
        @pl.when(cid < 97)
        def _():
            pltpu.sync_copy(acc.at[pl.ds(cid * 512, 512)], rows.at[pl.ds(0, 512)])
            pltpu.sync_copy(rows.at[pl.ds(0, 512)],
                            out_hbm.at[core].at[pl.ds(cid * 512, 512)])
        @pl.when(cid == 97)
        def _():
            pltpu.sync_copy(acc.at[pl.ds(97 * 512, 336)], rows.at[pl.ds(0, 336)])
            pltpu.sync_copy(rows.at[pl.ds(0, 336)],
                            out_hbm.at[core].at[pl.ds(97 * 512, 336)])


def _agg_pass(D, g, src2, dst2, nw2):
    CHD = 4 if D == 32 else 8
    return pl.kernel(
        functools.partial(_agg_body, D, CHD),
        out_type=jax.ShapeDtypeStruct((NC, N, D), jnp.float32),
        mesh=_mesh(),
        compiler_params=pltpu.CompilerParams(needs_layout_passes=False, use_tc_tiling_on_sc=False),
        scratch_types=[
            pltpu.VMEM_SHARED((NPAD, D), jnp.float32),
            pltpu.VMEM((CHD * EM, D), jnp.float32),
            pltpu.VMEM((CHD, EM), jnp.int32),
            pltpu.VMEM((CHD, EM), jnp.int32),
            pltpu.VMEM((CHD, EM), jnp.float32),
            pltpu.SemaphoreType.DMA,
        ],
    )(g, src2, dst2, nw2)


def _prep_body(d0_ref, d1_ref, x0_ref, x1_ref, ce_ref, se_ref, g_ref, dis_ref):
    deg = d0_ref[0, 0, :] + d1_ref[0, 0, :] + 1.0
    dis = jnp.where(deg > 0, lax.rsqrt(jnp.maximum(deg, 1e-12)), 0.0)
    x0 = x0_ref[0, 0, :]
    x1 = x1_ref[0, 0, :]
    oh0 = (x0[:, None] == lax.broadcasted_iota(jnp.int32, (BLK, 200), 1))
    oh1 = (x1[:, None] == lax.broadcasted_iota(jnp.int32, (BLK, 100), 1))
    ce = jnp.dot(oh0.astype(jnp.float32), ce_ref[...],
                 preferred_element_type=jnp.float32)
    se = jnp.dot(oh1.astype(jnp.float32), se_ref[...],
                 preferred_element_type=jnp.float32)
    h0 = jnp.concatenate([ce, se], axis=1)
    g_ref[...] = h0 * dis[:, None]
    dis_ref[0, 0, :] = dis


def _prep_pass(d0, d1, x0r, x1r, CE, SE):
    return pl.pallas_call(
        _prep_body,
        grid=(GRID,),
        in_specs=[
            pl.BlockSpec((1, 1, BLK), lambda i: (i, 0, 0)),
            pl.BlockSpec((1, 1, BLK), lambda i: (i, 0, 0)),
            pl.BlockSpec((1, 1, BLK), lambda i: (i, 0, 0)),
            pl.BlockSpec((1, 1, BLK), lambda i: (i, 0, 0)),
            pl.BlockSpec((200, 16), lambda i: (0, 0)),
            pl.BlockSpec((100, 16), lambda i: (0, 0)),
        ],
        out_specs=[
            pl.BlockSpec((BLK, 32), lambda i: (i, 0)),
            pl.BlockSpec((1, 1, BLK), lambda i: (i, 0, 0)),
        ],
        out_shape=[
            jax.ShapeDtypeStruct((N, 32), jnp.float32),
            jax.ShapeDtypeStruct((GRID, 1, BLK), jnp.float32),
        ],
    )(d0, d1, x0r, x1r, CE, SE)


def _mid_body(p1_ref, g_ref, dis_ref, w1_ref, b1_ref, w2_ref, g2_ref):
    dis = dis_ref[0, 0, :]
    agg1 = p1_ref[0] + p1_ref[1] + g_ref[...] * dis[:, None]
    h1 = jnp.maximum(
        jnp.dot(agg1, w1_ref[...], preferred_element_type=jnp.float32)
        + b1_ref[...], 0.0)
    m2 = jnp.dot(h1, w2_ref[...], preferred_element_type=jnp.float32)
    g2_ref[...] = m2 * dis[:, None]


def _mid_pass(p1, g, dis3, W1, b1r, W2):
    return pl.pallas_call(
        _mid_body,
        grid=(GRID,),
        in_specs=[
            pl.BlockSpec((NC, BLK, 32), lambda i: (0, i, 0)),
            pl.BlockSpec((BLK, 32), lambda i: (i, 0)),
            pl.BlockSpec((1, 1, BLK), lambda i: (i, 0, 0)),
            pl.BlockSpec((32, 64), lambda i: (0, 0)),
            pl.BlockSpec((1, 64), lambda i: (0, 0)),
            pl.BlockSpec((64, 16), lambda i: (0, 0)),
        ],
        out_specs=pl.BlockSpec((BLK, 16), lambda i: (i, 0)),
        out_shape=jax.ShapeDtypeStruct((N, 16), jnp.float32),
    )(p1, g, dis3, W1, b1r, W2)


def _final_body(p2_ref, g2_ref, dis_ref, b2_ref, out_ref):
    dis = dis_ref[0, 0, :]
    out_ref[...] = (p2_ref[0] + p2_ref[1] + g2_ref[...] * dis[:, None]
                    + b2_ref[...])


def _final_pass(p2, g2, dis3, b2r):
    return pl.pallas_call(
        _final_body,
        grid=(GRID,),
        in_specs=[
            pl.BlockSpec((NC, BLK, 16), lambda i: (0, i, 0)),
            pl.BlockSpec((BLK, 16), lambda i: (i, 0)),
            pl.BlockSpec((1, 1, BLK), lambda i: (i, 0, 0)),
            pl.BlockSpec((1, 16), lambda i: (0, 0)),
        ],
        out_specs=pl.BlockSpec((BLK, 16), lambda i: (i, 0)),
        out_shape=jax.ShapeDtypeStruct((N, 16), jnp.float32),
    )(p2, g2, dis3, b2r)


def kernel(x, edge_index, edge_attr, CE, SE, W1, b1, W2, b2):
    padi = jnp.zeros((EP - E,), jnp.int32)
    padf = jnp.zeros((EP - E,), jnp.float32)
    src2 = jnp.concatenate([edge_index[0], padi]).reshape(ER, EM)
    dst2 = jnp.concatenate([edge_index[1], padi]).reshape(ER, EM)
    ew2 = jnp.concatenate([edge_attr, padf]).reshape(ER, EM)
    x0r = x[:, 0].reshape(GRID, 1, BLK)
    x1r = x[:, 1].reshape(GRID, 1, BLK)

    degp = _deg_pass(dst2, ew2).reshape(NC, N)
    d0 = degp[0].reshape(GRID, 1, BLK)
    d1 = degp[1].reshape(GRID, 1, BLK)
    g, dis3 = _prep_pass(d0, d1, x0r, x1r, CE, SE)
    dis = dis3.reshape(N)
    nw2 = _normw_pass(dis, dst2, ew2)
    p1 = _agg_pass(32, g, src2, dst2, nw2)
    g2 = _mid_pass(p1, g, dis3, W1, b1.reshape(1, 64), W2)
    p2 = _agg_pass(16, g2, src2, dst2, nw2)
    return _final_pass(p2, g2, dis3, b2.reshape(1, 16))


# speedup vs baseline: 35.3175x; 1.7550x over previous
import functools

import jax
import jax.numpy as jnp
from jax import lax
from jax.experimental import pallas as pl
from jax.experimental.pallas import tpu as pltpu, tpu_sc as plsc

N = 50000
E = 800000
EM = 128
ER = 6256
EP = ER * EM
CH = 8
NCHUNK = ER // CH
NPAD = 49 * 1024
NC, NS = 2, 16
NW = NC * NS
BLK = 1000
GRID = N // BLK

_mesh = functools.partial(
    plsc.VectorSubcoreMesh, core_axis_name="c", subcore_axis_name="s",
    num_cores=NC, num_subcores=NS)


def _zero16():
    return jnp.zeros((16,), jnp.float32)


def _deg_body(dst_hbm, ew_hbm, out_hbm, acc, zbuf, dbuf, wbuf, sem):
    core = lax.axis_index("c")
    sid = lax.axis_index("s")
    wid = sid * NC + core

    def zb(i, _):
        zbuf[pl.ds(i * 16, 16)] = _zero16()
        return 0
    lax.fori_loop(0, 64, zb, 0)
    for z in range(4):
        cid = sid + z * NS
        @pl.when(cid < NPAD // 1024)
        def _():
            pltpu.sync_copy(zbuf, acc.at[pl.ds(cid * 1024, 1024)])
    plsc.subcore_barrier()

    def chunk(c, _):
        cid = c * NW + wid
        @pl.when(cid < NCHUNK)
        def _():
            r0 = cid * CH
            pltpu.sync_copy(dst_hbm.at[pl.ds(r0, CH)], dbuf)
            pltpu.sync_copy(ew_hbm.at[pl.ds(r0, CH)], wbuf)
            for j in range(CH):
                pltpu.sync_copy(wbuf.at[j], acc.at[dbuf.at[j]], add=True)
        return 0
    lax.fori_loop(0, (NCHUNK + NW - 1) // NW, chunk, 0)
    plsc.subcore_barrier()

    for z in range(4):
        cid = sid + z * NS
        @pl.when(cid < 48)
        def _():
            pltpu.sync_copy(acc.at[pl.ds(cid * 1024, 1024)], zbuf)
            pltpu.sync_copy(zbuf, out_hbm.at[pl.ds(core * N + cid * 1024, 1024)])
        @pl.when(cid == 48)
        def _():
            pltpu.sync_copy(acc.at[pl.ds(48 * 1024, 848)], zbuf.at[pl.ds(0, 848)])
            pltpu.sync_copy(zbuf.at[pl.ds(0, 848)],
                            out_hbm.at[pl.ds(core * N + 48 * 1024, 848)])


def _deg_pass(dst2, ew2):
    return pl.kernel(
        _deg_body,
        out_type=jax.ShapeDtypeStruct((NC * N,), jnp.float32),
        mesh=_mesh(),
        compiler_params=pltpu.CompilerParams(needs_layout_passes=False, use_tc_tiling_on_sc=False),
        scratch_types=[
            pltpu.VMEM_SHARED((NPAD,), jnp.float32),
            pltpu.VMEM((1024,), jnp.float32),
            pltpu.VMEM((CH, EM), jnp.int32),
            pltpu.VMEM((CH, EM), jnp.float32),
            pltpu.SemaphoreType.DMA,
        ],
    )(dst2, ew2)


def _normw_body(dis_hbm, dst_hbm, ew_hbm, out_hbm, dis_v, dbuf, wbuf, obuf):
    core = lax.axis_index("c")
    sid = lax.axis_index("s")
    wid = sid * NC + core

    pltpu.sync_copy(dis_hbm, dis_v)

    def chunk(c, _):
        cid = c * NW + wid
        @pl.when(cid < NCHUNK)
        def _():
            r0 = cid * CH
            pltpu.sync_copy(dst_hbm.at[pl.ds(r0, CH)], dbuf)
            pltpu.sync_copy(ew_hbm.at[pl.ds(r0, CH)], wbuf)

            def scale16(q, _):
                j = q // (EM // 16)
                o = (q % (EM // 16)) * 16
                d16 = dbuf[j, pl.ds(o, 16)]
                dval = plsc.load_gather(dis_v, [d16])
                obuf[j, pl.ds(o, 16)] = wbuf[j, pl.ds(o, 16)] * dval
                return 0
            lax.fori_loop(0, CH * EM // 16, scale16, 0)
            pltpu.sync_copy(obuf, out_hbm.at[pl.ds(r0, CH)])
        return 0
    lax.fori_loop(0, (NCHUNK + NW - 1) // NW, chunk, 0)


def _normw_pass(dis, dst2, ew2):
    return pl.kernel(
        _normw_body,
        out_type=jax.ShapeDtypeStruct((ER, EM), jnp.float32),
        mesh=_mesh(),
        compiler_params=pltpu.CompilerParams(needs_layout_passes=False, use_tc_tiling_on_sc=False),
        scratch_types=[
            pltpu.VMEM((N,), jnp.float32),
            pltpu.VMEM((CH, EM), jnp.int32),
            pltpu.VMEM((CH, EM), jnp.float32),
            pltpu.VMEM((CH, EM), jnp.float32),
        ],
    )(dis, dst2, ew2)


def _agg_body(D, CHD, g_hbm, src_hbm, dst_hbm, nw_hbm, out_hbm,
              acc, rows, sbuf, dbuf, wbuf, lsem, gsem0, gsem1, ssem0, ssem1):
    core = lax.axis_index("c")
    sid = lax.axis_index("s")
    wid = sid * NC + core
    nh = D // 16
    zrows = CHD * EM
    nz = NPAD // zrows
    nchunk = ER // CHD
    gsems = (gsem0, gsem1)
    ssems = (ssem0, ssem1)
    dnums = lax.GatherDimensionNumbers(
        offset_dims=(), collapsed_slice_dims=(0,), start_index_map=(0,))

    def zr(r, _):
        for h in range(nh):
            rows[r, pl.ds(h * 16, 16)] = _zero16()
        return 0
    lax.fori_loop(0, zrows, zr, 0)
    for z in range((nz + NS - 1) // NS):
        cid = sid + z * NS
        @pl.when(cid < nz)
        def _():
            pltpu.sync_copy(rows, acc.at[pl.ds(cid * zrows, zrows)])
    plsc.subcore_barrier()

    def chunk(c, _):
        cid = c * NW + wid
        @pl.when(cid < nchunk)
        def _():
            r0 = cid * CHD
            dl = [pltpu.async_copy(src_hbm.at[pl.ds(r0, CHD)], sbuf, lsem),
                  pltpu.async_copy(dst_hbm.at[pl.ds(r0, CHD)], dbuf, lsem),
                  pltpu.async_copy(nw_hbm.at[pl.ds(r0, CHD)], wbuf, lsem)]
            for d in dl:
                d.wait()

            def gath(j):
                return pltpu.async_copy(
                    g_hbm.at[sbuf.at[j]], rows.at[pl.ds(j * EM, EM)],
                    gsems[j % 2])

            def scat(j):
                return pltpu.async_copy(
                    rows.at[pl.ds(j * EM, EM)], acc.at[dbuf.at[j]],
                    ssems[j % 2], add=True)

            gd = [None] * CHD
            sd = [None] * CHD
            gd[0] = gath(0)
            if CHD > 1:
                gd[1] = gath(1)
            for j in range(CHD):
                gd[j].wait()
                if j + 2 < CHD:
                    gd[j + 2] = gath(j + 2)

                def grp(q, _):
                    jj = q // (EM // 16)
                    o = (q % (EM // 16)) * 16
                    s16 = wbuf[jj, pl.ds(o, 16)]
                    base = jj * EM + o
                    for u in range(16):
                        spl = lax.gather(
                            s16, jnp.full((16, 1), u, jnp.int32), dnums, (1,),
                            mode=lax.GatherScatterMode.PROMISE_IN_BOUNDS)
                        e = base + u
                        for h in range(nh):
                            rows[e, pl.ds(h * 16, 16)] = (
                                rows[e, pl.ds(h * 16, 16)] * spl)
                    return 0
                lax.fori_loop(j * (EM // 16), (j + 1) * (EM // 16), grp, 0)

                if j - 2 >= 0:
                    sd[j - 2].wait()
                sd[j] = scat(j)
            for j in range(max(CHD - 2, 0), CHD):
                sd[j].wait()
        return 0
    lax.fori_loop(0, (nchunk + NW - 1) // NW, chunk, 0)
    plsc.subcore_barrier()

    for z in range(7):
        cid = sid + z * NS
        @pl.when(cid < 97)
        def _():
            pltpu.sync_copy(acc.at[pl.ds(cid * 512, 512)], rows.at[pl.ds(0, 512)])
            pltpu.sync_copy(rows.at[pl.ds(0, 512)],
                            out_hbm.at[core].at[pl.ds(cid * 512, 512)])
        @pl.when(cid == 97)
        def _():
            pltpu.sync_copy(acc.at[pl.ds(97 * 512, 336)], rows.at[pl.ds(0, 336)])
            pltpu.sync_copy(rows.at[pl.ds(0, 336)],
                            out_hbm.at[core].at[pl.ds(97 * 512, 336)])


def _agg_pass(D, g, src2, dst2, nw2):
    CHD = 4 if D == 32 else 16
    return pl.kernel(
        functools.partial(_agg_body, D, CHD),
        out_type=jax.ShapeDtypeStruct((NC, N, D), jnp.float32),
        mesh=_mesh(),
        compiler_params=pltpu.CompilerParams(needs_layout_passes=False, use_tc_tiling_on_sc=False),
        scratch_types=[
            pltpu.VMEM_SHARED((NPAD, D), jnp.float32),
            pltpu.VMEM((CHD * EM, D), jnp.float32),
            pltpu.VMEM((CHD, EM), jnp.int32),
            pltpu.VMEM((CHD, EM), jnp.int32),
            pltpu.VMEM((CHD, EM), jnp.float32),
            pltpu.SemaphoreType.DMA,
            pltpu.SemaphoreType.DMA,
            pltpu.SemaphoreType.DMA,
            pltpu.SemaphoreType.DMA,
            pltpu.SemaphoreType.DMA,
        ],
    )(g, src2, dst2, nw2)


def _prep_body(d0_ref, d1_ref, x0_ref, x1_ref, ce_ref, se_ref, g_ref, dis_ref):
    deg = d0_ref[0, 0, :] + d1_ref[0, 0, :] + 1.0
    dis = jnp.where(deg > 0, lax.rsqrt(jnp.maximum(deg, 1e-12)), 0.0)
    x0 = x0_ref[0, 0, :]
    x1 = x1_ref[0, 0, :]
    oh0 = (x0[:, None] == lax.broadcasted_iota(jnp.int32, (BLK, 200), 1))
    oh1 = (x1[:, None] == lax.broadcasted_iota(jnp.int32, (BLK, 100), 1))
    ce = jnp.dot(oh0.astype(jnp.float32), ce_ref[...],
                 preferred_element_type=jnp.float32)
    se = jnp.dot(oh1.astype(jnp.float32), se_ref[...],
                 preferred_element_type=jnp.float32)
    h0 = jnp.concatenate([ce, se], axis=1)
    g_ref[...] = h0 * dis[:, None]
    dis_ref[0, 0, :] = dis


def _prep_pass(d0, d1, x0r, x1r, CE, SE):
    return pl.pallas_call(
        _prep_body,
        grid=(GRID,),
        in_specs=[
            pl.BlockSpec((1, 1, BLK), lambda i: (i, 0, 0)),
            pl.BlockSpec((1, 1, BLK), lambda i: (i, 0, 0)),
            pl.BlockSpec((1, 1, BLK), lambda i: (i, 0, 0)),
            pl.BlockSpec((1, 1, BLK), lambda i: (i, 0, 0)),
            pl.BlockSpec((200, 16), lambda i: (0, 0)),
            pl.BlockSpec((100, 16), lambda i: (0, 0)),
        ],
        out_specs=[
            pl.BlockSpec((BLK, 32), lambda i: (i, 0)),
            pl.BlockSpec((1, 1, BLK), lambda i: (i, 0, 0)),
        ],
        out_shape=[
            jax.ShapeDtypeStruct((N, 32), jnp.float32),
            jax.ShapeDtypeStruct((GRID, 1, BLK), jnp.float32),
        ],
    )(d0, d1, x0r, x1r, CE, SE)


def _mid_body(p1_ref, g_ref, dis_ref, w1_ref, b1_ref, w2_ref, g2_ref):
    dis = dis_ref[0, 0, :]
    agg1 = p1_ref[0] + p1_ref[1] + g_ref[...] * dis[:, None]
    h1 = jnp.maximum(
        jnp.dot(agg1, w1_ref[...], preferred_element_type=jnp.float32)
        + b1_ref[...], 0.0)
    m2 = jnp.dot(h1, w2_ref[...], preferred_element_type=jnp.float32)
    g2_ref[...] = m2 * dis[:, None]


def _mid_pass(p1, g, dis3, W1, b1r, W2):
    return pl.pallas_call(
        _mid_body,
        grid=(GRID,),
        in_specs=[
            pl.BlockSpec((NC, BLK, 32), lambda i: (0, i, 0)),
            pl.BlockSpec((BLK, 32), lambda i: (i, 0)),
            pl.BlockSpec((1, 1, BLK), lambda i: (i, 0, 0)),
            pl.BlockSpec((32, 64), lambda i: (0, 0)),
            pl.BlockSpec((1, 64), lambda i: (0, 0)),
            pl.BlockSpec((64, 16), lambda i: (0, 0)),
        ],
        out_specs=pl.BlockSpec((BLK, 16), lambda i: (i, 0)),
        out_shape=jax.ShapeDtypeStruct((N, 16), jnp.float32),
    )(p1, g, dis3, W1, b1r, W2)


def _final_body(p2_ref, g2_ref, dis_ref, b2_ref, out_ref):
    dis = dis_ref[0, 0, :]
    out_ref[...] = (p2_ref[0] + p2_ref[1] + g2_ref[...] * dis[:, None]
                    + b2_ref[...])


def _final_pass(p2, g2, dis3, b2r):
    return pl.pallas_call(
        _final_body,
        grid=(GRID,),
        in_specs=[
            pl.BlockSpec((NC, BLK, 16), lambda i: (0, i, 0)),
            pl.BlockSpec((BLK, 16), lambda i: (i, 0)),
            pl.BlockSpec((1, 1, BLK), lambda i: (i, 0, 0)),
            pl.BlockSpec((1, 16), lambda i: (0, 0)),
        ],
        out_specs=pl.BlockSpec((BLK, 16), lambda i: (i, 0)),
        out_shape=jax.ShapeDtypeStruct((N, 16), jnp.float32),
    )(p2, g2, dis3, b2r)


def kernel(x, edge_index, edge_attr, CE, SE, W1, b1, W2, b2):
    padi = jnp.zeros((EP - E,), jnp.int32)
    padf = jnp.zeros((EP - E,), jnp.float32)
    src2 = jnp.concatenate([edge_index[0], padi]).reshape(ER, EM)
    dst2 = jnp.concatenate([edge_index[1], padi]).reshape(ER, EM)
    ew2 = jnp.concatenate([edge_attr, padf]).reshape(ER, EM)
    x0r = x[:, 0].reshape(GRID, 1, BLK)
    x1r = x[:, 1].reshape(GRID, 1, BLK)

    degp = _deg_pass(dst2, ew2).reshape(NC, N)
    d0 = degp[0].reshape(GRID, 1, BLK)
    d1 = degp[1].reshape(GRID, 1, BLK)
    g, dis3 = _prep_pass(d0, d1, x0r, x1r, CE, SE)
    dis = dis3.reshape(N)
    nw2 = _normw_pass(dis, dst2, ew2)
    p1 = _agg_pass(32, g, src2, dst2, nw2)
    g2 = _mid_pass(p1, g, dis3, W1, b1.reshape(1, 64), W2)
    p2 = _agg_pass(16, g2, src2, dst2, nw2)
    return _final_pass(p2, g2, dis3, b2.reshape(1, 16))


# speedup vs baseline: 35.6524x; 1.0095x over previous
import functools

import jax
import jax.numpy as jnp
from jax import lax
from jax.experimental import pallas as pl
from jax.experimental.pallas import tpu as pltpu, tpu_sc as plsc

N = 50000
E = 800000
EM = 128
ER = 6256
EP = ER * EM
CH = 8
NCHUNK = ER // CH
NPAD = 49 * 1024
NC, NS = 2, 16
NW = NC * NS
BLK = 1000
GRID = N // BLK

_mesh = functools.partial(
    plsc.VectorSubcoreMesh, core_axis_name="c", subcore_axis_name="s",
    num_cores=NC, num_subcores=NS)


def _zero16():
    return jnp.zeros((16,), jnp.float32)


def _deg_body(dst_hbm, ew_hbm, out_hbm, acc, zbuf, dbuf, wbuf, sem):
    core = lax.axis_index("c")
    sid = lax.axis_index("s")
    wid = sid * NC + core

    def zb(i, _):
        zbuf[pl.ds(i * 16, 16)] = _zero16()
        return 0
    lax.fori_loop(0, 64, zb, 0)
    for z in range(4):
        cid = sid + z * NS
        @pl.when(cid < NPAD // 1024)
        def _():
            pltpu.sync_copy(zbuf, acc.at[pl.ds(cid * 1024, 1024)])
    plsc.subcore_barrier()

    def chunk(c, _):
        cid = c * NW + wid
        @pl.when(cid < NCHUNK)
        def _():
            r0 = cid * CH
            pltpu.sync_copy(dst_hbm.at[pl.ds(r0, CH)], dbuf)
            pltpu.sync_copy(ew_hbm.at[pl.ds(r0, CH)], wbuf)
            for j in range(CH):
                pltpu.sync_copy(wbuf.at[j], acc.at[dbuf.at[j]], add=True)
        return 0
    lax.fori_loop(0, (NCHUNK + NW - 1) // NW, chunk, 0)
    plsc.subcore_barrier()

    for z in range(4):
        cid = sid + z * NS
        @pl.when(cid < 48)
        def _():
            pltpu.sync_copy(acc.at[pl.ds(cid * 1024, 1024)], zbuf)
            pltpu.sync_copy(zbuf, out_hbm.at[pl.ds(core * N + cid * 1024, 1024)])
        @pl.when(cid == 48)
        def _():
            pltpu.sync_copy(acc.at[pl.ds(48 * 1024, 848)], zbuf.at[pl.ds(0, 848)])
            pltpu.sync_copy(zbuf.at[pl.ds(0, 848)],
                            out_hbm.at[pl.ds(core * N + 48 * 1024, 848)])


def _deg_pass(dst2, ew2):
    return pl.kernel(
        _deg_body,
        out_type=jax.ShapeDtypeStruct((NC * N,), jnp.float32),
        mesh=_mesh(),
        compiler_params=pltpu.CompilerParams(needs_layout_passes=False, use_tc_tiling_on_sc=False),
        scratch_types=[
            pltpu.VMEM_SHARED((NPAD,), jnp.float32),
            pltpu.VMEM((1024,), jnp.float32),
            pltpu.VMEM((CH, EM), jnp.int32),
            pltpu.VMEM((CH, EM), jnp.float32),
            pltpu.SemaphoreType.DMA,
        ],
    )(dst2, ew2)


def _normw_body(dis_hbm, dst_hbm, ew_hbm, out_hbm, dis_v, dbuf, wbuf, obuf):
    core = lax.axis_index("c")
    sid = lax.axis_index("s")
    wid = sid * NC + core

    pltpu.sync_copy(dis_hbm, dis_v)

    def chunk(c, _):
        cid = c * NW + wid
        @pl.when(cid < NCHUNK)
        def _():
            r0 = cid * CH
            pltpu.sync_copy(dst_hbm.at[pl.ds(r0, CH)], dbuf)
            pltpu.sync_copy(ew_hbm.at[pl.ds(r0, CH)], wbuf)

            def scale16(q, _):
                j = q // (EM // 16)
                o = (q % (EM // 16)) * 16
                d16 = dbuf[j, pl.ds(o, 16)]
                dval = plsc.load_gather(dis_v, [d16])
                obuf[j, pl.ds(o, 16)] = wbuf[j, pl.ds(o, 16)] * dval
                return 0
            lax.fori_loop(0, CH * EM // 16, scale16, 0)
            pltpu.sync_copy(obuf, out_hbm.at[pl.ds(r0, CH)])
        return 0
    lax.fori_loop(0, (NCHUNK + NW - 1) // NW, chunk, 0)


def _normw_pass(dis, dst2, ew2):
    return pl.kernel(
        _normw_body,
        out_type=jax.ShapeDtypeStruct((ER, EM), jnp.float32),
        mesh=_mesh(),
        compiler_params=pltpu.CompilerParams(needs_layout_passes=False, use_tc_tiling_on_sc=False),
        scratch_types=[
            pltpu.VMEM((N,), jnp.float32),
            pltpu.VMEM((CH, EM), jnp.int32),
            pltpu.VMEM((CH, EM), jnp.float32),
            pltpu.VMEM((CH, EM), jnp.float32),
        ],
    )(dis, dst2, ew2)


def _agg_body(D, CHD, g_hbm, src_hbm, dst_hbm, nw_hbm, out_hbm,
              acc, rows, sbuf, dbuf, wbuf, lsem, gsem0, gsem1, ssem0, ssem1):
    core = lax.axis_index("c")
    sid = lax.axis_index("s")
    wid = sid * NC + core
    nh = D // 16
    zrows = CHD * EM
    nz = NPAD // zrows
    nchunk = ER // CHD
    gsems = (gsem0, gsem1)
    ssems = (ssem0, ssem1)
    dnums = lax.GatherDimensionNumbers(
        offset_dims=(), collapsed_slice_dims=(0,), start_index_map=(0,))

    def zr(r, _):
        for h in range(nh):
            rows[r, pl.ds(h * 16, 16)] = _zero16()
        return 0
    lax.fori_loop(0, 512, zr, 0)
    for z in range(7):
        cid = sid + z * NS
        @pl.when(cid < NPAD // 512)
        def _():
            pltpu.sync_copy(rows.at[pl.ds(0, 512)],
                            acc.at[pl.ds(cid * 512, 512)])
    plsc.subcore_barrier()

    def chunk(c, _):
        cid = c * NW + wid
        @pl.when(cid < nchunk)
        def _():
            r0 = cid * CHD
            dl = [pltpu.async_copy(src_hbm.at[pl.ds(r0, CHD)], sbuf, lsem),
                  pltpu.async_copy(dst_hbm.at[pl.ds(r0, CHD)], dbuf, lsem),
                  pltpu.async_copy(nw_hbm.at[pl.ds(r0, CHD)], wbuf, lsem)]
            for d in dl:
                d.wait()

            def gath(j):
                return pltpu.async_copy(
                    g_hbm.at[sbuf.at[j]], rows.at[pl.ds(j * EM, EM)],
                    gsems[j % 2])

            def scat(j):
                return pltpu.async_copy(
                    rows.at[pl.ds(j * EM, EM)], acc.at[dbuf.at[j]],
                    ssems[j % 2], add=True)

            gd = [None] * CHD
            sd = [None] * CHD
            gd[0] = gath(0)
            if CHD > 1:
                gd[1] = gath(1)
            for j in range(CHD):
                gd[j].wait()
                if j + 2 < CHD:
                    gd[j + 2] = gath(j + 2)

                def grp(q, _):
                    jj = q // (EM // 16)
                    o = (q % (EM // 16)) * 16
                    s16 = wbuf[jj, pl.ds(o, 16)]
                    base = jj * EM + o
                    for u in range(16):
                        spl = lax.gather(
                            s16, jnp.full((16, 1), u, jnp.int32), dnums, (1,),
                            mode=lax.GatherScatterMode.PROMISE_IN_BOUNDS)
                        e = base + u
                        for h in range(nh):
                            rows[e, pl.ds(h * 16, 16)] = (
                                rows[e, pl.ds(h * 16, 16)] * spl)
                    return 0
                lax.fori_loop(j * (EM // 16), (j + 1) * (EM // 16), grp, 0)

                if j - 2 >= 0:
                    sd[j - 2].wait()
                sd[j] = scat(j)
            for j in range(max(CHD - 2, 0), CHD):
                sd[j].wait()
        return 0
    lax.fori_loop(0, (nchunk + NW - 1) // NW, chunk, 0)
    plsc.subcore_barrier()

    for z in range(7):
        cid = sid + z * NS
        @pl.when(cid < 97)
        def _():
            pltpu.sync_copy(acc.at[pl.ds(cid * 512, 512)], rows.at[pl.ds(0, 512)])
            pltpu.sync_copy(rows.at[pl.ds(0, 512)],
                            out_hbm.at[core].at[pl.ds(cid * 512, 512)])
        @pl.when(cid == 97)
        def _():
            pltpu.sync_copy(acc.at[pl.ds(97 * 512, 336)], rows.at[pl.ds(0, 336)])
            pltpu.sync_copy(rows.at[pl.ds(0, 336)],
                            out_hbm.at[core].at[pl.ds(97 * 512, 336)])


def _agg_pass(D, g, src2, dst2, nw2):
    CHD = 4 if D == 32 else 16
    return pl.kernel(
        functools.partial(_agg_body, D, CHD),
        out_type=jax.ShapeDtypeStruct((NC, N, D), jnp.float32),
        mesh=_mesh(),
        compiler_params=pltpu.CompilerParams(needs_layout_passes=False, use_tc_tiling_on_sc=False),
        scratch_types=[
            pltpu.VMEM_SHARED((NPAD, D), jnp.float32),
            pltpu.VMEM((CHD * EM, D), jnp.float32),
            pltpu.VMEM((CHD, EM), jnp.int32),
            pltpu.VMEM((CHD, EM), jnp.int32),
            pltpu.VMEM((CHD, EM), jnp.float32),
            pltpu.SemaphoreType.DMA,
            pltpu.SemaphoreType.DMA,
            pltpu.SemaphoreType.DMA,
            pltpu.SemaphoreType.DMA,
            pltpu.SemaphoreType.DMA,
        ],
    )(g, src2, dst2, nw2)


def _prep_body(d0_ref, d1_ref, x0_ref, x1_ref, ce_ref, se_ref, g_ref, dis_ref):
    deg = d0_ref[0, 0, :] + d1_ref[0, 0, :] + 1.0
    dis = jnp.where(deg > 0, lax.rsqrt(jnp.maximum(deg, 1e-12)), 0.0)
    x0 = x0_ref[0, 0, :]
    x1 = x1_ref[0, 0, :]
    oh0 = (x0[:, None] == lax.broadcasted_iota(jnp.int32, (BLK, 200), 1))
    oh1 = (x1[:, None] == lax.broadcasted_iota(jnp.int32, (BLK, 100), 1))
    ce = jnp.dot(oh0.astype(jnp.float32), ce_ref[...],
                 preferred_element_type=jnp.float32)
    se = jnp.dot(oh1.astype(jnp.float32), se_ref[...],
                 preferred_element_type=jnp.float32)
    h0 = jnp.concatenate([ce, se], axis=1)
    g_ref[...] = h0 * dis[:, None]
    dis_ref[0, 0, :] = dis


def _prep_pass(d0, d1, x0r, x1r, CE, SE):
    return pl.pallas_call(
        _prep_body,
        grid=(GRID,),
        in_specs=[
            pl.BlockSpec((1, 1, BLK), lambda i: (i, 0, 0)),
            pl.BlockSpec((1, 1, BLK), lambda i: (i, 0, 0)),
            pl.BlockSpec((1, 1, BLK), lambda i: (i, 0, 0)),
            pl.BlockSpec((1, 1, BLK), lambda i: (i, 0, 0)),
            pl.BlockSpec((200, 16), lambda i: (0, 0)),
            pl.BlockSpec((100, 16), lambda i: (0, 0)),
        ],
        out_specs=[
            pl.BlockSpec((BLK, 32), lambda i: (i, 0)),
            pl.BlockSpec((1, 1, BLK), lambda i: (i, 0, 0)),
        ],
        out_shape=[
            jax.ShapeDtypeStruct((N, 32), jnp.float32),
            jax.ShapeDtypeStruct((GRID, 1, BLK), jnp.float32),
        ],
    )(d0, d1, x0r, x1r, CE, SE)


def _mid_body(p1_ref, g_ref, dis_ref, w1_ref, b1_ref, w2_ref, g2_ref):
    dis = dis_ref[0, 0, :]
    agg1 = p1_ref[0] + p1_ref[1] + g_ref[...] * dis[:, None]
    h1 = jnp.maximum(
        jnp.dot(agg1, w1_ref[...], preferred_element_type=jnp.float32)
        + b1_ref[...], 0.0)
    m2 = jnp.dot(h1, w2_ref[...], preferred_element_type=jnp.float32)
    g2_ref[...] = m2 * dis[:, None]


def _mid_pass(p1, g, dis3, W1, b1r, W2):
    return pl.pallas_call(
        _mid_body,
        grid=(GRID,),
        in_specs=[
            pl.BlockSpec((NC, BLK, 32), lambda i: (0, i, 0)),
            pl.BlockSpec((BLK, 32), lambda i: (i, 0)),
            pl.BlockSpec((1, 1, BLK), lambda i: (i, 0, 0)),
            pl.BlockSpec((32, 64), lambda i: (0, 0)),
            pl.BlockSpec((1, 64), lambda i: (0, 0)),
            pl.BlockSpec((64, 16), lambda i: (0, 0)),
        ],
        out_specs=pl.BlockSpec((BLK, 16), lambda i: (i, 0)),
        out_shape=jax.ShapeDtypeStruct((N, 16), jnp.float32),
    )(p1, g, dis3, W1, b1r, W2)


def _final_body(p2_ref, g2_ref, dis_ref, b2_ref, out_ref):
    dis = dis_ref[0, 0, :]
    out_ref[...] = (p2_ref[0] + p2_ref[1] + g2_ref[...] * dis[:, None]
                    + b2_ref[...])


def _final_pass(p2, g2, dis3, b2r):
    return pl.pallas_call(
        _final_body,
        grid=(GRID,),
        in_specs=[
            pl.BlockSpec((NC, BLK, 16), lambda i: (0, i, 0)),
            pl.BlockSpec((BLK, 16), lambda i: (i, 0)),
            pl.BlockSpec((1, 1, BLK), lambda i: (i, 0, 0)),
            pl.BlockSpec((1, 16), lambda i: (0, 0)),
        ],
        out_specs=pl.BlockSpec((BLK, 16), lambda i: (i, 0)),
        out_shape=jax.ShapeDtypeStruct((N, 16), jnp.float32),
    )(p2, g2, dis3, b2r)


def kernel(x, edge_index, edge_attr, CE, SE, W1, b1, W2, b2):
    padi = jnp.zeros((EP - E,), jnp.int32)
    padf = jnp.zeros((EP - E,), jnp.float32)
    src2 = jnp.concatenate([edge_index[0], padi]).reshape(ER, EM)
    dst2 = jnp.concatenate([edge_index[1], padi]).reshape(ER, EM)
    ew2 = jnp.concatenate([edge_attr, padf]).reshape(ER, EM)
    x0r = x[:, 0].reshape(GRID, 1, BLK)
    x1r = x[:, 1].reshape(GRID, 1, BLK)

    degp = _deg_pass(dst2, ew2).reshape(NC, N)
    d0 = degp[0].reshape(GRID, 1, BLK)
    d1 = degp[1].reshape(GRID, 1, BLK)
    g, dis3 = _prep_pass(d0, d1, x0r, x1r, CE, SE)
    dis = dis3.reshape(N)
    nw2 = _normw_pass(dis, dst2, ew2)
    p1 = _agg_pass(32, g, src2, dst2, nw2)
    g2 = _mid_pass(p1, g, dis3, W1, b1.reshape(1, 64), W2)
    p2 = _agg_pass(16, g2, src2, dst2, nw2)
    return _final_pass(p2, g2, dis3, b2.reshape(1, 16))
